# baseline (speedup 1.0000x reference)
import dataclasses
import functools

import jax
import jax.numpy as jnp
from jax import lax
from jax.experimental import pallas as pl
from jax.experimental.pallas import tpu as pltpu
from jax.experimental.pallas import tpu_sc as plsc

_SC_CORES = 2
_SC_SUBCORES = 16
_VOCAB_BLK = 5888
_LANES = 16


def _gather_t_sc(et, idx):
    E, V = et.shape
    B = idx.shape[0]
    nw = _SC_CORES * _SC_SUBCORES
    rows_per_w = E // nw
    mesh = plsc.VectorSubcoreMesh(core_axis_name="c", subcore_axis_name="s")
    cp = pltpu.CompilerParams()
    if "needs_layout_passes" in pltpu.CompilerParams.__dataclass_fields__:
        cp = dataclasses.replace(cp, needs_layout_passes=False)

    @functools.partial(
        pl.kernel,
        mesh=mesh,
        compiler_params=cp,
        out_type=jax.ShapeDtypeStruct((E + 1, B), jnp.float32),
        scratch_types=[
            pltpu.VMEM((B,), jnp.int32),
            pltpu.VMEM((1, V), jnp.float32),
            pltpu.VMEM((1, B), jnp.float32),
            pltpu.SemaphoreType.DMA,
            pltpu.SemaphoreType.DMA,
        ],
    )
    def gather_kernel(et_hbm, idx_hbm, out_hbm, idx_v, row_v, out_v, sem_i, sem_r):
        wid = lax.axis_index("s") * _SC_CORES + lax.axis_index("c")
        row1 = row_v.at[0]
        out1 = out_v.at[0]
        pltpu.async_copy(idx_hbm, idx_v, sem_i).wait()
        for p in range(rows_per_w):
            f = wid * rows_per_w + p
            pltpu.async_copy(et_hbm.at[pl.ds(f, 1)], row_v, sem_r).wait()

            @pl.loop(0, B, step=_LANES)
            def _(c):
                iv = idx_v[pl.ds(c, _LANES)]
                out1[pl.ds(c, _LANES)] = plsc.load_gather(row1, [iv])

            pltpu.sync_copy(out_v, out_hbm.at[pl.ds(f, 1)])

        @pl.when(wid == 0)
        def _():
            @pl.loop(0, B, step=_LANES)
            def _(c):
                out1[pl.ds(c, _LANES)] = jnp.full((_LANES,), 1.0, jnp.float32)

            pltpu.sync_copy(out_v, out_hbm.at[pl.ds(E, 1)])

    return gather_kernel(et, idx)


def _project_t_kernel(e_ref, w_ref, b_ref, out_ref):
    w_aug = jnp.concatenate([w_ref[...], b_ref[...]], axis=0)
    out_ref[...] = jax.lax.dot_general(
        w_aug.astype(jnp.bfloat16), e_ref[...].astype(jnp.bfloat16),
        dimension_numbers=(((0,), (0,)), ((), ())),
        preferred_element_type=jnp.float32,
    )


def kernel(context_word, emb_table, W, b):
    B = context_word.shape[0]
    V, E = W.shape
    idx = context_word.astype(jnp.int32)
    et = emb_table.T
    wt = W.T
    e_aug = _gather_t_sc(et, idx)
    b2 = b.reshape(1, V)
    out_t = pl.pallas_call(
        _project_t_kernel,
        grid=(pl.cdiv(V, _VOCAB_BLK),),
        in_specs=[
            pl.BlockSpec((E + 1, B), lambda j: (0, 0)),
            pl.BlockSpec((E, _VOCAB_BLK), lambda j: (0, j)),
            pl.BlockSpec((1, _VOCAB_BLK), lambda j: (0, j)),
        ],
        out_specs=pl.BlockSpec((_VOCAB_BLK, B), lambda j: (j, 0)),
        out_shape=jax.ShapeDtypeStruct((V, B), jnp.float32),
        compiler_params=pltpu.CompilerParams(
            dimension_semantics=("arbitrary",),
        ),
    )(e_aug, wt, b2)
    return out_t.T


# speedup vs baseline: 1.0083x; 1.0083x over previous
import dataclasses
import functools

import jax
import jax.numpy as jnp
from jax import lax
from jax.experimental import pallas as pl
from jax.experimental.pallas import tpu as pltpu
from jax.experimental.pallas import tpu_sc as plsc

_SC_CORES = 2
_SC_SUBCORES = 16
_VOCAB_BLK = 4352
_LANES = 16


def _gather_t_sc(et, idx):
    E, V = et.shape
    B = idx.shape[0]
    nw = _SC_CORES * _SC_SUBCORES
    rows_per_w = E // nw
    mesh = plsc.VectorSubcoreMesh(core_axis_name="c", subcore_axis_name="s")
    cp = pltpu.CompilerParams()
    if "needs_layout_passes" in pltpu.CompilerParams.__dataclass_fields__:
        cp = dataclasses.replace(cp, needs_layout_passes=False)

    @functools.partial(
        pl.kernel,
        mesh=mesh,
        compiler_params=cp,
        out_type=jax.ShapeDtypeStruct((E + 1, B), jnp.float32),
        scratch_types=[
            pltpu.VMEM((B,), jnp.int32),
            pltpu.VMEM((1, V), jnp.float32),
            pltpu.VMEM((1, B), jnp.float32),
            pltpu.SemaphoreType.DMA,
            pltpu.SemaphoreType.DMA,
        ],
    )
    def gather_kernel(et_hbm, idx_hbm, out_hbm, idx_v, row_v, out_v, sem_i, sem_r):
        wid = lax.axis_index("s") * _SC_CORES + lax.axis_index("c")
        row1 = row_v.at[0]
        out1 = out_v.at[0]
        pltpu.async_copy(idx_hbm, idx_v, sem_i).wait()
        for p in range(rows_per_w):
            f = wid * rows_per_w + p
            pltpu.async_copy(et_hbm.at[pl.ds(f, 1)], row_v, sem_r).wait()

            @pl.loop(0, B, step=_LANES)
            def _(c):
                iv = idx_v[pl.ds(c, _LANES)]
                out1[pl.ds(c, _LANES)] = plsc.load_gather(row1, [iv])

            pltpu.sync_copy(out_v, out_hbm.at[pl.ds(f, 1)])

        @pl.when(wid == 0)
        def _():
            @pl.loop(0, B, step=_LANES)
            def _(c):
                out1[pl.ds(c, _LANES)] = jnp.full((_LANES,), 1.0, jnp.float32)

            pltpu.sync_copy(out_v, out_hbm.at[pl.ds(E, 1)])

    return gather_kernel(et, idx)


def _project_t_kernel(e_ref, w_ref, b_ref, out_ref):
    w_aug = jnp.concatenate([w_ref[...], b_ref[...]], axis=0)
    out_ref[...] = jax.lax.dot_general(
        w_aug.astype(jnp.bfloat16), e_ref[...].astype(jnp.bfloat16),
        dimension_numbers=(((0,), (0,)), ((), ())),
        preferred_element_type=jnp.float32,
    )


def kernel(context_word, emb_table, W, b):
    B = context_word.shape[0]
    V, E = W.shape
    idx = context_word.astype(jnp.int32)
    et = emb_table.T
    wt = W.T
    e_aug = _gather_t_sc(et, idx)
    b2 = b.reshape(1, V)
    out_t = pl.pallas_call(
        _project_t_kernel,
        grid=(pl.cdiv(V, _VOCAB_BLK),),
        in_specs=[
            pl.BlockSpec((E + 1, B), lambda j: (0, 0)),
            pl.BlockSpec((E, _VOCAB_BLK), lambda j: (0, j)),
            pl.BlockSpec((1, _VOCAB_BLK), lambda j: (0, j)),
        ],
        out_specs=pl.BlockSpec((_VOCAB_BLK, B), lambda j: (j, 0)),
        out_shape=jax.ShapeDtypeStruct((V, B), jnp.float32),
        compiler_params=pltpu.CompilerParams(
            dimension_semantics=("arbitrary",),
        ),
    )(e_aug, wt, b2)
    return out_t.T


# speedup vs baseline: 1.0125x; 1.0042x over previous
import dataclasses
import functools

import jax
import jax.numpy as jnp
from jax import lax
from jax.experimental import pallas as pl
from jax.experimental.pallas import tpu as pltpu
from jax.experimental.pallas import tpu_sc as plsc

_SC_CORES = 2
_SC_SUBCORES = 16
_VOCAB_BLK = 4096
_LANES = 16


def _gather_t_sc(et, idx):
    E, V = et.shape
    B = idx.shape[0]
    nw = _SC_CORES * _SC_SUBCORES
    rows_per_w = E // nw
    mesh = plsc.VectorSubcoreMesh(core_axis_name="c", subcore_axis_name="s")
    cp = pltpu.CompilerParams()
    if "needs_layout_passes" in pltpu.CompilerParams.__dataclass_fields__:
        cp = dataclasses.replace(cp, needs_layout_passes=False)

    @functools.partial(
        pl.kernel,
        mesh=mesh,
        compiler_params=cp,
        out_type=jax.ShapeDtypeStruct((E + 1, B), jnp.float32),
        scratch_types=[
            pltpu.VMEM((B,), jnp.int32),
            pltpu.VMEM((1, V), jnp.float32),
            pltpu.VMEM((1, B), jnp.float32),
            pltpu.SemaphoreType.DMA,
            pltpu.SemaphoreType.DMA,
        ],
    )
    def gather_kernel(et_hbm, idx_hbm, out_hbm, idx_v, row_v, out_v, sem_i, sem_r):
        wid = lax.axis_index("s") * _SC_CORES + lax.axis_index("c")
        row1 = row_v.at[0]
        out1 = out_v.at[0]
        idx_cp = pltpu.async_copy(idx_hbm, idx_v, sem_i)
        f0 = wid * rows_per_w
        row_cp = pltpu.async_copy(et_hbm.at[pl.ds(f0, 1)], row_v, sem_r)
        idx_cp.wait()
        for p in range(rows_per_w):
            f = wid * rows_per_w + p
            row_cp.wait()

            @pl.loop(0, B, step=_LANES)
            def _(c):
                iv = idx_v[pl.ds(c, _LANES)]
                out1[pl.ds(c, _LANES)] = plsc.load_gather(row1, [iv])

            if p + 1 < rows_per_w:
                pltpu.sync_copy(out_v, out_hbm.at[pl.ds(f, 1)])
                row_cp = pltpu.async_copy(et_hbm.at[pl.ds(f + 1, 1)], row_v, sem_r)
            else:
                pltpu.sync_copy(out_v, out_hbm.at[pl.ds(f, 1)])

        @pl.when(wid == 0)
        def _():
            @pl.loop(0, B, step=_LANES)
            def _(c):
                out1[pl.ds(c, _LANES)] = jnp.full((_LANES,), 1.0, jnp.float32)

            pltpu.sync_copy(out_v, out_hbm.at[pl.ds(E, 1)])

    return gather_kernel(et, idx)


def _project_t_kernel(e_ref, w_ref, b_ref, out_ref):
    w_aug = jnp.concatenate([w_ref[...], b_ref[...]], axis=0)
    out_ref[...] = jax.lax.dot_general(
        w_aug.astype(jnp.bfloat16), e_ref[...].astype(jnp.bfloat16),
        dimension_numbers=(((0,), (0,)), ((), ())),
        preferred_element_type=jnp.float32,
    )


def kernel(context_word, emb_table, W, b):
    B = context_word.shape[0]
    V, E = W.shape
    idx = context_word.astype(jnp.int32)
    et = emb_table.T
    wt = W.T
    e_aug = _gather_t_sc(et, idx)
    b2 = b.reshape(1, V)
    out_t = pl.pallas_call(
        _project_t_kernel,
        grid=(pl.cdiv(V, _VOCAB_BLK),),
        in_specs=[
            pl.BlockSpec((E + 1, B), lambda j: (0, 0)),
            pl.BlockSpec((E, _VOCAB_BLK), lambda j: (0, j)),
            pl.BlockSpec((1, _VOCAB_BLK), lambda j: (0, j)),
        ],
        out_specs=pl.BlockSpec((_VOCAB_BLK, B), lambda j: (j, 0)),
        out_shape=jax.ShapeDtypeStruct((V, B), jnp.float32),
        compiler_params=pltpu.CompilerParams(
            dimension_semantics=("arbitrary",),
        ),
    )(e_aug, wt, b2)
    return out_t.T


# speedup vs baseline: 1.0128x; 1.0003x over previous
import dataclasses
import functools

import jax
import jax.numpy as jnp
from jax import lax
from jax.experimental import pallas as pl
from jax.experimental.pallas import tpu as pltpu
from jax.experimental.pallas import tpu_sc as plsc

_SC_CORES = 2
_SC_SUBCORES = 16
_VOCAB_BLK = 4096
_LANES = 16


def _gather_t_sc(et, idx):
    E, V = et.shape
    B = idx.shape[0]
    nw = _SC_CORES * _SC_SUBCORES
    rows_per_w = E // nw
    mesh = plsc.VectorSubcoreMesh(core_axis_name="c", subcore_axis_name="s")
    cp = pltpu.CompilerParams()
    if "needs_layout_passes" in pltpu.CompilerParams.__dataclass_fields__:
        cp = dataclasses.replace(cp, needs_layout_passes=False)

    half = 50048

    @functools.partial(
        pl.kernel,
        mesh=mesh,
        compiler_params=cp,
        out_type=jax.ShapeDtypeStruct((E + 1, B), jnp.float32),
        scratch_types=[
            pltpu.VMEM((B,), jnp.int32),
            pltpu.VMEM((1, half), jnp.float32),
            pltpu.VMEM((1, V - half), jnp.float32),
            pltpu.VMEM((1, B), jnp.float32),
            pltpu.SemaphoreType.DMA,
            pltpu.SemaphoreType.DMA,
            pltpu.SemaphoreType.DMA,
        ],
    )
    def gather_kernel(et_hbm, idx_hbm, out_hbm, idx_v, buf_a, buf_b, out_v,
                      sem_i, sem_a, sem_b):
        wid = lax.axis_index("s") * _SC_CORES + lax.axis_index("c")
        buf2d = (buf_a, buf_b)
        bufs = (buf_a.at[0], buf_b.at[0])
        sems = (sem_a, sem_b)
        lens = (half, V - half)
        out1 = out_v.at[0]
        f0 = wid * rows_per_w
        idx_cp = pltpu.async_copy(idx_hbm, idx_v, sem_i)

        def task_dma(t):
            r, h = t // 2, t % 2
            return pltpu.async_copy(
                et_hbm.at[pl.ds(f0 + r, 1), pl.ds(h * half, lens[h])],
                buf2d[h], sems[h])

        cps = {0: task_dma(0)}
        idx_cp.wait()
        n_tasks = 2 * rows_per_w
        for t in range(n_tasks):
            cps.pop(t).wait()
            if t + 1 < n_tasks:
                cps[t + 1] = task_dma(t + 1)
            h = t % 2
            src = bufs[h]

            if h == 0:

                @pl.loop(0, B, step=_LANES)
                def _(c):
                    iv = idx_v[pl.ds(c, _LANES)]
                    ivc = jnp.minimum(iv, half - 1)
                    out1[pl.ds(c, _LANES)] = plsc.load_gather(src, [ivc])

            else:

                @pl.loop(0, B, step=_LANES)
                def _(c):
                    iv = idx_v[pl.ds(c, _LANES)]
                    m = iv >= half
                    ivc = jnp.maximum(iv - half, 0)
                    vals = plsc.load_gather(src, [ivc])
                    cur = out1[pl.ds(c, _LANES)]
                    out1[pl.ds(c, _LANES)] = jnp.where(m, vals, cur)

                pltpu.sync_copy(out_v, out_hbm.at[pl.ds(f0 + t // 2, 1)])

        @pl.when(wid == 0)
        def _():
            @pl.loop(0, B, step=_LANES)
            def _(c):
                out1[pl.ds(c, _LANES)] = jnp.full((_LANES,), 1.0, jnp.float32)

            pltpu.sync_copy(out_v, out_hbm.at[pl.ds(E, 1)])

    return gather_kernel(et, idx)


def _project_t_kernel(e_ref, w_ref, b_ref, out_ref):
    w_aug = jnp.concatenate([w_ref[...], b_ref[...]], axis=0)
    out_ref[...] = jax.lax.dot_general(
        w_aug.astype(jnp.bfloat16), e_ref[...].astype(jnp.bfloat16),
        dimension_numbers=(((0,), (0,)), ((), ())),
        preferred_element_type=jnp.float32,
    )


def kernel(context_word, emb_table, W, b):
    B = context_word.shape[0]
    V, E = W.shape
    idx = context_word.astype(jnp.int32)
    et = emb_table.T
    wt = W.T
    e_aug = _gather_t_sc(et, idx)
    b2 = b.reshape(1, V)
    out_t = pl.pallas_call(
        _project_t_kernel,
        grid=(pl.cdiv(V, _VOCAB_BLK),),
        in_specs=[
            pl.BlockSpec((E + 1, B), lambda j: (0, 0)),
            pl.BlockSpec((E, _VOCAB_BLK), lambda j: (0, j)),
            pl.BlockSpec((1, _VOCAB_BLK), lambda j: (0, j)),
        ],
        out_specs=pl.BlockSpec((_VOCAB_BLK, B), lambda j: (j, 0)),
        out_shape=jax.ShapeDtypeStruct((V, B), jnp.float32),
        compiler_params=pltpu.CompilerParams(
            dimension_semantics=("arbitrary",),
        ),
    )(e_aug, wt, b2)
    return out_t.T
